# baseline (device time: 82342 ns/iter reference)
import jax
import jax.numpy as jnp
from jax import lax
from jax.experimental import pallas as pl
from jax.experimental.pallas import tpu as pltpu

N_DEV = 4
K = 4


def kernel(x):
    m, n = x.shape
    ch = m // N_DEV
    half = ch // 2
    piece = half // K
    n_hops = 2 * (N_DEV - 1)
    rs_hops = N_DEV - 1

    def body(x_ref, out_ref, stage_ref, send_sems, recv_sems):
        d = lax.axis_index("i")
        left = lax.rem(d + N_DEV - 1, N_DEV)
        right = lax.rem(d + 1, N_DEV)

        barrier_sem = pltpu.get_barrier_semaphore()
        for nbr in (left, right):
            pl.semaphore_signal(
                barrier_sem, inc=1,
                device_id=(nbr,), device_id_type=pl.DeviceIdType.MESH,
            )
        pl.semaphore_wait(barrier_sem, 2)

        def c_send_of(dirn, h):
            if h < rs_hops:
                return (
                    lax.rem(d + N_DEV - h, N_DEV)
                    if dirn == 0
                    else lax.rem(d + h, N_DEV)
                )
            t = h - rs_hops
            return (
                lax.rem(d + 1 + N_DEV - t, N_DEV)
                if dirn == 0
                else lax.rem(d + N_DEV - 1 + t, N_DEV)
            )

        def c_recv_of(dirn, h):
            if h < rs_hops:
                return (
                    lax.rem(d + N_DEV - h - 1, N_DEV)
                    if dirn == 0
                    else lax.rem(d + h + 1, N_DEV)
                )
            t = h - rs_hops
            return (
                lax.rem(d + N_DEV - t, N_DEV)
                if dirn == 0
                else lax.rem(d + t, N_DEV)
            )

        def row_of(chunk_idx, dirn, j):
            return chunk_idx * ch + dirn * half + j * piece

        sent = []

        def send(dirn, h, j):
            tgt = right if dirn == 0 else left
            row = row_of(c_send_of(dirn, h), dirn, j)
            src = out_ref.at[pl.ds(row, piece), :]
            if h < rs_hops:
                dst = stage_ref.at[dirn, h, j]
            else:
                dst = out_ref.at[pl.ds(row, piece), :]
            rdma = pltpu.make_async_remote_copy(
                src_ref=src,
                dst_ref=dst,
                send_sem=send_sems.at[dirn, h, j],
                recv_sem=recv_sems.at[dirn, h, j],
                device_id=(tgt,),
                device_id_type=pl.DeviceIdType.MESH,
            )
            rdma.start()
            sent.append(rdma)

        def wait_recv(dirn, h, j):
            if h < rs_hops:
                dst = stage_ref.at[dirn, h, j]
            else:
                row = row_of(c_recv_of(dirn, h), dirn, j)
                dst = out_ref.at[pl.ds(row, piece), :]
            rdma = pltpu.make_async_remote_copy(
                src_ref=dst,
                dst_ref=dst,
                send_sem=send_sems.at[dirn, h, j],
                recv_sem=recv_sems.at[dirn, h, j],
                device_id=(left if dirn == 0 else right,),
                device_id_type=pl.DeviceIdType.MESH,
            )
            rdma.wait_recv()

        out_ref[pl.ds(d * ch, ch), :] = (
            x_ref[pl.ds(d * ch, ch), :].astype(jnp.bfloat16)
        )
        for j in range(K):
            for dirn in range(2):
                send(dirn, 0, j)
        for off in range(1, N_DEV):
            c = lax.rem(d + off, N_DEV)
            out_ref[pl.ds(c * ch, ch), :] = (
                x_ref[pl.ds(c * ch, ch), :].astype(jnp.bfloat16)
            )

        for h in range(n_hops):
            for j in range(K):
                for dirn in range(2):
                    wait_recv(dirn, h, j)
                    if h < rs_hops:
                        row = row_of(c_recv_of(dirn, h), dirn, j)
                        out_ref[pl.ds(row, piece), :] = (
                            out_ref[pl.ds(row, piece), :]
                            + stage_ref[dirn, h, j]
                        )
                    if h + 1 < n_hops:
                        send(dirn, h + 1, j)

        for rdma in sent:
            rdma.wait_send()

    return pl.pallas_call(
        body,
        out_shape=jax.ShapeDtypeStruct((m, n), jnp.bfloat16),
        in_specs=[pl.BlockSpec(memory_space=pltpu.VMEM)],
        out_specs=pl.BlockSpec(memory_space=pltpu.VMEM),
        scratch_shapes=[
            pltpu.VMEM((2, rs_hops, K, piece, n), jnp.bfloat16),
            pltpu.SemaphoreType.DMA((2, n_hops, K)),
            pltpu.SemaphoreType.DMA((2, n_hops, K)),
        ],
        compiler_params=pltpu.CompilerParams(collective_id=0),
    )(x)


# device time: 81839 ns/iter; 1.0061x vs baseline; 1.0061x over previous
import jax
import jax.numpy as jnp
from jax import lax
from jax.experimental import pallas as pl
from jax.experimental.pallas import tpu as pltpu

N_DEV = 4
K = 2


def kernel(x):
    m, n = x.shape
    ch = m // N_DEV
    half = ch // 2
    piece = half // K
    n_hops = 2 * (N_DEV - 1)
    rs_hops = N_DEV - 1

    def body(x_ref, out_ref, stage_ref, send_sems, recv_sems):
        d = lax.axis_index("i")
        left = lax.rem(d + N_DEV - 1, N_DEV)
        right = lax.rem(d + 1, N_DEV)

        barrier_sem = pltpu.get_barrier_semaphore()
        for nbr in (left, right):
            pl.semaphore_signal(
                barrier_sem, inc=1,
                device_id=(nbr,), device_id_type=pl.DeviceIdType.MESH,
            )
        pl.semaphore_wait(barrier_sem, 2)

        def c_send_of(dirn, h):
            if h < rs_hops:
                return (
                    lax.rem(d + N_DEV - h, N_DEV)
                    if dirn == 0
                    else lax.rem(d + h, N_DEV)
                )
            t = h - rs_hops
            return (
                lax.rem(d + 1 + N_DEV - t, N_DEV)
                if dirn == 0
                else lax.rem(d + N_DEV - 1 + t, N_DEV)
            )

        def c_recv_of(dirn, h):
            if h < rs_hops:
                return (
                    lax.rem(d + N_DEV - h - 1, N_DEV)
                    if dirn == 0
                    else lax.rem(d + h + 1, N_DEV)
                )
            t = h - rs_hops
            return (
                lax.rem(d + N_DEV - t, N_DEV)
                if dirn == 0
                else lax.rem(d + t, N_DEV)
            )

        def row_of(chunk_idx, dirn, j):
            return chunk_idx * ch + dirn * half + j * piece

        sent = []

        def send(dirn, h, j):
            tgt = right if dirn == 0 else left
            row = row_of(c_send_of(dirn, h), dirn, j)
            src = out_ref.at[pl.ds(row, piece), :]
            if h < rs_hops:
                dst = stage_ref.at[dirn, h, j]
            else:
                dst = out_ref.at[pl.ds(row, piece), :]
            rdma = pltpu.make_async_remote_copy(
                src_ref=src,
                dst_ref=dst,
                send_sem=send_sems.at[dirn, h, j],
                recv_sem=recv_sems.at[dirn, h, j],
                device_id=(tgt,),
                device_id_type=pl.DeviceIdType.MESH,
            )
            rdma.start()
            sent.append(rdma)

        def wait_recv(dirn, h, j):
            if h < rs_hops:
                dst = stage_ref.at[dirn, h, j]
            else:
                row = row_of(c_recv_of(dirn, h), dirn, j)
                dst = out_ref.at[pl.ds(row, piece), :]
            rdma = pltpu.make_async_remote_copy(
                src_ref=dst,
                dst_ref=dst,
                send_sem=send_sems.at[dirn, h, j],
                recv_sem=recv_sems.at[dirn, h, j],
                device_id=(left if dirn == 0 else right,),
                device_id_type=pl.DeviceIdType.MESH,
            )
            rdma.wait_recv()

        out_ref[pl.ds(d * ch, ch), :] = (
            x_ref[pl.ds(d * ch, ch), :].astype(jnp.bfloat16)
        )
        for j in range(K):
            for dirn in range(2):
                send(dirn, 0, j)
        for off in range(1, N_DEV):
            c = lax.rem(d + off, N_DEV)
            out_ref[pl.ds(c * ch, ch), :] = (
                x_ref[pl.ds(c * ch, ch), :].astype(jnp.bfloat16)
            )

        for h in range(n_hops):
            for j in range(K):
                for dirn in range(2):
                    wait_recv(dirn, h, j)
                    if h < rs_hops:
                        row = row_of(c_recv_of(dirn, h), dirn, j)
                        out_ref[pl.ds(row, piece), :] = (
                            out_ref[pl.ds(row, piece), :]
                            + stage_ref[dirn, h, j]
                        )
                    if h + 1 < n_hops:
                        send(dirn, h + 1, j)

        for rdma in sent:
            rdma.wait_send()

    return pl.pallas_call(
        body,
        out_shape=jax.ShapeDtypeStruct((m, n), jnp.bfloat16),
        in_specs=[pl.BlockSpec(memory_space=pltpu.VMEM)],
        out_specs=pl.BlockSpec(memory_space=pltpu.VMEM),
        scratch_shapes=[
            pltpu.VMEM((2, rs_hops, K, piece, n), jnp.bfloat16),
            pltpu.SemaphoreType.DMA((2, n_hops, K)),
            pltpu.SemaphoreType.DMA((2, n_hops, K)),
        ],
        compiler_params=pltpu.CompilerParams(collective_id=0),
    )(x)
